# baseline (speedup 1.0000x reference)
import jax
import jax.numpy as jnp
from jax import lax
from jax.experimental import pallas as pl
from jax.experimental.pallas import tpu as pltpu
from jax.experimental.pallas import tpu_sc as plsc

N_NODES = 10000
D_IN = 128
H = 128
D_OUT = 64
N_EDGES = 320000

NC = 2
NS = 16
EPT = N_EDGES // NS
B = 80
NCHUNK = EPT // B
DEG_CHUNKS = NCHUNK // NC
NBUF = 5
NOUTER = NCHUNK // NBUF

RCHUNK = 80
N_RCHUNKS = N_NODES // RCHUNK
RCHUNKS_PER_TILE = -(-N_RCHUNKS // NS)


def _sc_aggregate(wh, with_deg):
  mesh = plsc.VectorSubcoreMesh(core_axis_name="c", subcore_axis_name="s")
  out_type = [jax.ShapeDtypeStruct((NC, N_NODES, wh), jnp.float32)]
  scratch = [
      pltpu.VMEM((NCHUNK, B), jnp.int32),
      pltpu.VMEM((NCHUNK, B), jnp.int32),
      pltpu.VMEM((NBUF, B, wh), jnp.float32),
      pltpu.VMEM((RCHUNK, wh), jnp.float32),
      pltpu.VMEM_SHARED((N_NODES, wh), jnp.float32),
      pltpu.SemaphoreType.DMA((NBUF,)),
      pltpu.SemaphoreType.DMA((NBUF,)),
  ]
  if with_deg:
    out_type.append(jax.ShapeDtypeStruct((NC, N_NODES, 16), jnp.float32))
    scratch += [
        pltpu.VMEM((B, 16), jnp.float32),
        pltpu.VMEM((RCHUNK, 16), jnp.float32),
        pltpu.VMEM_SHARED((N_NODES, 16), jnp.float32),
        pltpu.SemaphoreType.DMA,
    ]

  def body(*refs):
    if with_deg:
      (feats, src3, dst3, z_w, z16, ones16, acc_out, deg_out,
       src_v, dst_v, rows_v, zbuf, acc_sh, gsem, ssem,
       ones_v, dzbuf, deg_sh, dsem) = refs
    else:
      (feats, src3, dst3, z_w, acc_out,
       src_v, dst_v, rows_v, zbuf, acc_sh, gsem, ssem) = refs

    cid = lax.axis_index("c")
    sid = lax.axis_index("s")

    def row_chunk_offsets():
      for j in range(RCHUNKS_PER_TILE):
        c = j * NS + sid
        off = pl.multiple_of(c * RCHUNK, RCHUNK)
        yield c, off

    pltpu.sync_copy(src3.at[sid], src_v)

    offset = cid.astype(jnp.int32)

    def add_off(k, carry):
      for j in range(B // 16):
        sl = pl.ds(j * 16, 16)
        src_v[k, sl] = src_v[k, sl] * 2 + offset
      return carry

    lax.fori_loop(0, NCHUNK, add_off, 0)

    for b in range(NBUF):
      pltpu.async_copy(feats.at[src_v.at[b]], rows_v.at[b], gsem.at[b])

    pltpu.sync_copy(dst3.at[sid], dst_v)
    pltpu.sync_copy(z_w, zbuf)
    if with_deg:
      pltpu.sync_copy(ones16, ones_v)
      pltpu.sync_copy(z16, dzbuf)

    for c, off in row_chunk_offsets():
      @pl.when(c < N_RCHUNKS)
      def _():
        pltpu.sync_copy(zbuf, acc_sh.at[pl.ds(off, RCHUNK)])
        if with_deg:
          pltpu.sync_copy(dzbuf, deg_sh.at[pl.ds(off, RCHUNK)])
    plsc.subcore_barrier()

    deg_lo = cid * DEG_CHUNKS
    deg_hi = deg_lo + DEG_CHUNKS

    def outer(g, carry):
      for b in range(NBUF):
        kk = g * NBUF + b
        pltpu.make_async_copy(feats.at[pl.ds(0, B)], rows_v.at[b],
                              gsem.at[b]).wait()
        pltpu.async_copy(rows_v.at[b], acc_sh.at[dst_v.at[kk]], ssem.at[b],
                         add=True)
        if with_deg:
          @pl.when(jnp.logical_and(kk >= deg_lo, kk < deg_hi))
          def _():
            @pl.when(kk > deg_lo)
            def _():
              pltpu.make_async_copy(ones16, ones_v, dsem).wait()
            pltpu.async_copy(ones_v, deg_sh.at[dst_v.at[kk]], dsem, add=True)

      @pl.when(g < NOUTER - 1)
      def _():
        for b in range(NBUF):
          kk = g * NBUF + b
          pltpu.make_async_copy(feats.at[pl.ds(0, B)], rows_v.at[b],
                                ssem.at[b]).wait()
          pltpu.async_copy(feats.at[src_v.at[kk + NBUF]], rows_v.at[b],
                           gsem.at[b])
      return carry

    lax.fori_loop(0, NOUTER, outer, 0)
    for b in range(NBUF):
      pltpu.make_async_copy(feats.at[pl.ds(0, B)], rows_v.at[b],
                            ssem.at[b]).wait()
    if with_deg:
      pltpu.make_async_copy(ones16, ones_v, dsem).wait()
    plsc.subcore_barrier()

    for c, off in row_chunk_offsets():
      @pl.when(c < N_RCHUNKS)
      def _():
        pltpu.sync_copy(acc_sh.at[pl.ds(off, RCHUNK)],
                        acc_out.at[cid, pl.ds(off, RCHUNK)])
        if with_deg:
          pltpu.sync_copy(deg_sh.at[pl.ds(off, RCHUNK)],
                          deg_out.at[cid, pl.ds(off, RCHUNK)])

  return pl.kernel(
      body, out_type=out_type, mesh=mesh, scratch_types=scratch,
      compiler_params=pltpu.CompilerParams(use_tc_tiling_on_sc=False))


def _tc_layer1(acc, deg, x, w1l, w1r, b1, w2l, w2r, b2):
  def body(acc_ref, deg_ref, x_ref, w1l_ref, w1r_ref, b1_ref,
           w2l_ref, w2r_ref, b2_ref, p2_ref, r2_ref):
    a = jnp.concatenate([acc_ref[0], acc_ref[1]], axis=-1)
    d = deg_ref[0, :, 0:1] + deg_ref[1, :, 0:1]
    mean = a / jnp.maximum(d, 1.0)
    h = mean @ w1l_ref[...] + x_ref[...] @ w1r_ref[...] + b1_ref[...][None, :]
    h = jnp.maximum(h, 0.0)
    p2_ref[...] = h @ w2l_ref[...]
    r2_ref[...] = h @ w2r_ref[...] + b2_ref[...][None, :]

  return pl.pallas_call(
      body,
      out_shape=[
          jax.ShapeDtypeStruct((N_NODES, D_OUT), jnp.float32),
          jax.ShapeDtypeStruct((N_NODES, D_OUT), jnp.float32),
      ],
  )(acc, deg, x, w1l, w1r, b1, w2l, w2r, b2)


def _tc_layer2(acc2, deg, r2):
  def body(acc_ref, deg_ref, r2_ref, out_ref):
    a = jnp.concatenate([acc_ref[0], acc_ref[1]], axis=-1)
    d = deg_ref[0, :, 0:1] + deg_ref[1, :, 0:1]
    out_ref[...] = a / jnp.maximum(d, 1.0) + r2_ref[...]

  return pl.pallas_call(
      body,
      out_shape=jax.ShapeDtypeStruct((N_NODES, D_OUT), jnp.float32),
  )(acc2, deg, r2)


@jax.jit
def kernel(x, edge_index, W1l, W1r, b1, W2l, W2r, b2):
  ei = edge_index.astype(jnp.int32)
  src3 = ei[0].reshape(NS, NCHUNK, B)
  dst3 = ei[1].reshape(NS, NCHUNK, B)
  z64 = jnp.zeros((RCHUNK, H // 2), jnp.float32)
  z32 = jnp.zeros((RCHUNK, D_OUT // 2), jnp.float32)
  z16 = jnp.zeros((RCHUNK, 16), jnp.float32)
  ones16 = jnp.ones((B, 16), jnp.float32)

  agg1 = _sc_aggregate(H // 2, True)
  acc1, deg = agg1(x.reshape(2 * N_NODES, H // 2), src3, dst3, z64, z16,
                   ones16)
  p2, r2 = _tc_layer1(acc1, deg, x, W1l, W1r, b1, W2l, W2r, b2)
  agg2 = _sc_aggregate(D_OUT // 2, False)
  (acc2,) = agg2(p2.reshape(2 * N_NODES, D_OUT // 2), src3, dst3, z32)
  return _tc_layer2(acc2, deg, r2)


# speedup vs baseline: 1.0216x; 1.0216x over previous
import jax
import jax.numpy as jnp
from jax import lax
from jax.experimental import pallas as pl
from jax.experimental.pallas import tpu as pltpu
from jax.experimental.pallas import tpu_sc as plsc

N_NODES = 10000
D_IN = 128
H = 128
D_OUT = 64
N_EDGES = 320000

NC = 2
NS = 16
EPT = N_EDGES // NS
B = 80
NCHUNK = EPT // B
DEG_CHUNKS = NCHUNK // NC
NBUF = 5
NOUTER = NCHUNK // NBUF

RCHUNK = 80
N_RCHUNKS = N_NODES // RCHUNK
RCHUNKS_PER_TILE = -(-N_RCHUNKS // NS)


def _sc_aggregate(wh, with_deg):
  mesh = plsc.VectorSubcoreMesh(core_axis_name="c", subcore_axis_name="s")
  out_type = [jax.ShapeDtypeStruct((NC, N_NODES, wh), jnp.float32)]
  scratch = [
      pltpu.VMEM((NCHUNK, B), jnp.int32),
      pltpu.VMEM((NCHUNK, B), jnp.int32),
      pltpu.VMEM((NBUF, B, wh), jnp.float32),
      pltpu.VMEM((RCHUNK, wh), jnp.float32),
      pltpu.VMEM_SHARED((N_NODES, wh), jnp.float32),
      pltpu.SemaphoreType.DMA((NBUF,)),
      pltpu.SemaphoreType.DMA((NBUF,)),
  ]
  if with_deg:
    out_type.append(jax.ShapeDtypeStruct((NC, N_NODES, 16), jnp.float32))
    scratch += [
        pltpu.VMEM((B, 16), jnp.float32),
        pltpu.VMEM((RCHUNK, 16), jnp.float32),
        pltpu.VMEM_SHARED((N_NODES, 16), jnp.float32),
        pltpu.SemaphoreType.DMA,
    ]

  def body(*refs):
    if with_deg:
      (feats, src3, dst3, z_w, z16, ones16, acc_out, deg_out,
       src_v, dst_v, rows_v, zbuf, acc_sh, gsem, ssem,
       ones_v, dzbuf, deg_sh, dsem) = refs
    else:
      (feats, src3, dst3, z_w, acc_out,
       src_v, dst_v, rows_v, zbuf, acc_sh, gsem, ssem) = refs

    cid = lax.axis_index("c")
    sid = lax.axis_index("s")

    def row_chunk_offsets():
      for j in range(RCHUNKS_PER_TILE):
        c = j * NS + sid
        off = pl.multiple_of(c * RCHUNK, RCHUNK)
        yield c, off

    pltpu.sync_copy(src3.at[sid], src_v)

    offset = cid.astype(jnp.int32)

    def add_off(k, carry):
      for j in range(B // 16):
        sl = pl.ds(j * 16, 16)
        src_v[k, sl] = src_v[k, sl] * 2 + offset
      return carry

    lax.fori_loop(0, NCHUNK, add_off, 0)

    for b in range(NBUF):
      pltpu.async_copy(feats.at[src_v.at[b]], rows_v.at[b], gsem.at[b])

    pltpu.sync_copy(dst3.at[sid], dst_v)
    pltpu.sync_copy(z_w, zbuf)
    if with_deg:
      pltpu.sync_copy(ones16, ones_v)
      pltpu.sync_copy(z16, dzbuf)

    for c, off in row_chunk_offsets():
      @pl.when(c < N_RCHUNKS)
      def _():
        pltpu.sync_copy(zbuf, acc_sh.at[pl.ds(off, RCHUNK)])
        if with_deg:
          pltpu.sync_copy(dzbuf, deg_sh.at[pl.ds(off, RCHUNK)])
    plsc.subcore_barrier()

    deg_lo = cid * DEG_CHUNKS
    deg_hi = deg_lo + DEG_CHUNKS

    def outer(g, carry):
      for b in range(NBUF):
        kk = g * NBUF + b
        pltpu.make_async_copy(feats.at[pl.ds(0, B)], rows_v.at[b],
                              gsem.at[b]).wait()
        pltpu.async_copy(rows_v.at[b], acc_sh.at[dst_v.at[kk]], ssem.at[b],
                         add=True)
        if with_deg:
          @pl.when(jnp.logical_and(kk >= deg_lo, kk < deg_hi))
          def _():
            @pl.when(kk > deg_lo)
            def _():
              pltpu.make_async_copy(ones16, ones_v, dsem).wait()
            pltpu.async_copy(ones_v, deg_sh.at[dst_v.at[kk]], dsem, add=True)

      @pl.when(g < NOUTER - 1)
      def _():
        for b in range(NBUF):
          kk = g * NBUF + b
          pltpu.make_async_copy(feats.at[pl.ds(0, B)], rows_v.at[b],
                                ssem.at[b]).wait()
          pltpu.async_copy(feats.at[src_v.at[kk + NBUF]], rows_v.at[b],
                           gsem.at[b])
      return carry

    lax.fori_loop(0, NOUTER, outer, 0)
    for b in range(NBUF):
      pltpu.make_async_copy(feats.at[pl.ds(0, B)], rows_v.at[b],
                            ssem.at[b]).wait()
    if with_deg:
      pltpu.make_async_copy(ones16, ones_v, dsem).wait()
    plsc.subcore_barrier()

    for c, off in row_chunk_offsets():
      @pl.when(c < N_RCHUNKS)
      def _():
        pltpu.sync_copy(acc_sh.at[pl.ds(off, RCHUNK)], zbuf)
        pltpu.sync_copy(zbuf, acc_out.at[cid, pl.ds(off, RCHUNK)])
        if with_deg:
          pltpu.sync_copy(deg_sh.at[pl.ds(off, RCHUNK)], dzbuf)
          pltpu.sync_copy(dzbuf, deg_out.at[cid, pl.ds(off, RCHUNK)])

  return pl.kernel(
      body, out_type=out_type, mesh=mesh, scratch_types=scratch,
      compiler_params=pltpu.CompilerParams(use_tc_tiling_on_sc=False))


def _tc_layer1(acc, deg, x, w1l, w1r, b1, w2l, w2r, b2):
  def body(acc_ref, deg_ref, x_ref, w1l_ref, w1r_ref, b1_ref,
           w2l_ref, w2r_ref, b2_ref, p2_ref, r2_ref):
    a = jnp.concatenate([acc_ref[0], acc_ref[1]], axis=-1)
    d = deg_ref[0, :, 0:1] + deg_ref[1, :, 0:1]
    mean = a / jnp.maximum(d, 1.0)
    h = mean @ w1l_ref[...] + x_ref[...] @ w1r_ref[...] + b1_ref[...][None, :]
    h = jnp.maximum(h, 0.0)
    p2_ref[...] = h @ w2l_ref[...]
    r2_ref[...] = h @ w2r_ref[...] + b2_ref[...][None, :]

  return pl.pallas_call(
      body,
      out_shape=[
          jax.ShapeDtypeStruct((N_NODES, D_OUT), jnp.float32),
          jax.ShapeDtypeStruct((N_NODES, D_OUT), jnp.float32),
      ],
  )(acc, deg, x, w1l, w1r, b1, w2l, w2r, b2)


def _tc_layer2(acc2, deg, r2):
  def body(acc_ref, deg_ref, r2_ref, out_ref):
    a = jnp.concatenate([acc_ref[0], acc_ref[1]], axis=-1)
    d = deg_ref[0, :, 0:1] + deg_ref[1, :, 0:1]
    out_ref[...] = a / jnp.maximum(d, 1.0) + r2_ref[...]

  return pl.pallas_call(
      body,
      out_shape=jax.ShapeDtypeStruct((N_NODES, D_OUT), jnp.float32),
  )(acc2, deg, r2)


@jax.jit
def kernel(x, edge_index, W1l, W1r, b1, W2l, W2r, b2):
  ei = edge_index.astype(jnp.int32)
  src3 = ei[0].reshape(NS, NCHUNK, B)
  dst3 = ei[1].reshape(NS, NCHUNK, B)
  z64 = jnp.zeros((RCHUNK, H // 2), jnp.float32)
  z32 = jnp.zeros((RCHUNK, D_OUT // 2), jnp.float32)
  z16 = jnp.zeros((RCHUNK, 16), jnp.float32)
  ones16 = jnp.ones((B, 16), jnp.float32)

  agg1 = _sc_aggregate(H // 2, True)
  acc1, deg = agg1(x.reshape(2 * N_NODES, H // 2), src3, dst3, z64, z16,
                   ones16)
  p2, r2 = _tc_layer1(acc1, deg, x, W1l, W1r, b1, W2l, W2r, b2)
  agg2 = _sc_aggregate(D_OUT // 2, False)
  (acc2,) = agg2(p2.reshape(2 * N_NODES, D_OUT // 2), src3, dst3, z32)
  return _tc_layer2(acc2, deg, r2)
